# Initial kernel scaffold; baseline (speedup 1.0000x reference)
import jax
import jax.numpy as jnp
from jax.experimental import pallas as pl


def kernel(img, W1, b1, W2, b2):
    raise NotImplementedError("write your pallas kernel here")


# speedup vs baseline: 6.1016x; 6.1016x over previous
import jax
import jax.numpy as jnp
from jax.experimental import pallas as pl
from jax.experimental.pallas import tpu as pltpu

_NUM_CLASSES = 10
_ITERS = 3
_N = 256
_M = 256
_NN = 254
_MM = 254
_IN_DIM = 15
_OUT_DIM = 16
_D_IN = 137
_ST = 14


def _q(x):
    return x.astype(jnp.bfloat16).astype(jnp.float32)


def _prep_body(W1_ref, W2_ref, w1q_ref, w2q_ref):
    w1q_ref[:] = _q(W1_ref[:])
    w2q_ref[:] = _q(W2_ref[:])


def _nca_body(img_ref, w1q_ref, w2q_ref, b1_ref, b2_ref,
              cls_ref, g_ref, st_ref, stq_ref, perc_ref, h_ref):
    f32 = jnp.float32
    st_ref[...] = jnp.zeros(st_ref.shape, f32)

    gxi = jax.lax.broadcasted_iota(jnp.int32, (_NN, _MM), 0)
    gyi = jax.lax.broadcasted_iota(jnp.int32, (_NN, _MM), 1)
    dxp = jnp.zeros((_NN, _MM), jnp.int32)
    dyp = jnp.zeros((_NN, _MM), jnp.int32)

    for t in range(_ITERS):
        if t == 0:
            for dx in range(3):
                for dy in range(3):
                    perc_ref[dx * 3 + dy, 0:_NN, 0:_MM] = _q(
                        img_ref[2 + dx:2 + dx + _NN, 2 + dy:2 + dy + _MM]
                    )
        else:
            offs = list(range(-t, t + 1))
            masks = []
            for ox in offs:
                mrow = dxp == ox
                for oy in offs:
                    masks.append(((mrow & (dyp == oy)).astype(f32), ox, oy))
            for dx in range(3):
                for dy in range(3):
                    acc = None
                    for m, ox, oy in masks:
                        term = m * img_ref[
                            2 + ox + dx:2 + ox + dx + _NN,
                            2 + oy + dy:2 + oy + dy + _MM,
                        ]
                        acc = term if acc is None else acc + term
                    perc_ref[dx * 3 + dy, 0:_NN, 0:_MM] = _q(acc)

        px = _q((gxi + dxp - _N // 2).astype(f32) * (1.0 / (_N // 2)))
        py = _q((gyi + dyp - _M // 2).astype(f32) * (1.0 / (_M // 2)))

        def hcalc(k, _, t=t):
            a = px * w1q_ref[_D_IN - 2, k] + py * w1q_ref[_D_IN - 1, k]

            def pacc(i, a):
                return a + w1q_ref[i * _IN_DIM, k] * perc_ref[i, 0:_NN, 0:_MM]

            a = jax.lax.fori_loop(0, 9, pacc, a)
            if t > 0:
                for dx in range(3):
                    for dy in range(3):
                        base = (dx * 3 + dy) * _IN_DIM + 1

                        def cacc(c, a, dx=dx, dy=dy, base=base):
                            return a + w1q_ref[base + c, k] * stq_ref[
                                dy, c, dx:dx + _NN, 0:_MM
                            ]

                        a = jax.lax.fori_loop(0, _ST, cacc, a)
            h_ref[k, 0:_NN, 0:_MM] = (a + b1_ref[0, k]).astype(jnp.bfloat16)
            return 0

        jax.lax.fori_loop(0, _D_IN, hcalc, 0)

        def gcalc(o, _):
            def kacc(k, a):
                return a + w2q_ref[k, o] * h_ref[k, 0:_NN, 0:_MM].astype(f32)

            g_ref[o] = jax.lax.fori_loop(
                0, _D_IN, kacc, jnp.full((_NN, _MM), b2_ref[0, o], f32)
            )
            return 0

        jax.lax.fori_loop(0, _OUT_DIM, gcalc, 0)

        if t < _ITERS - 1:
            thr = 0.0007
            a14 = g_ref[_OUT_DIM - 2]
            a15 = g_ref[_OUT_DIM - 1]
            ax = jnp.where(a14 < -thr, -1, jnp.where(a14 > thr, 1, 0))
            ay = jnp.where(a15 < -thr, -1, jnp.where(a15 > thr, 1, 0))
            dxp = jnp.clip(gxi + dxp + ax.astype(jnp.int32), 0, _N - 3) - gxi
            dyp = jnp.clip(gyi + dyp + ay.astype(jnp.int32), 0, _M - 3) - gyi

        if t < _ITERS - 1:

            def supd(c, _):
                st_ref[c, 1:1 + _NN, 1:1 + _MM] = (
                    st_ref[c, 1:1 + _NN, 1:1 + _MM] + g_ref[c]
                )
                stq_ref[0, c] = _q(st_ref[c])
                stq_ref[1, c, :, 0:_M - 1] = _q(st_ref[c, :, 1:_M])
                stq_ref[2, c, :, 0:_M - 2] = _q(st_ref[c, :, 2:_M])
                return 0

            jax.lax.fori_loop(0, _ST, supd, 0)
        else:

            def supd_last(c, _):
                st_ref[c, 1:1 + _NN, 1:1 + _MM] = (
                    st_ref[c, 1:1 + _NN, 1:1 + _MM] + g_ref[c]
                )
                return 0

            jax.lax.fori_loop(0, _ST, supd_last, 0)

    def cwrite(k, _):
        cls_ref[k] = st_ref[_ST - _NUM_CLASSES + k, 1:1 + _NN, 1:1 + _MM]
        return 0

    jax.lax.fori_loop(0, _NUM_CLASSES, cwrite, 0)


_PREP_CALL = pl.pallas_call(
    _prep_body,
    out_shape=[
        jax.ShapeDtypeStruct((_D_IN, _D_IN), jnp.float32),
        jax.ShapeDtypeStruct((_D_IN, _OUT_DIM), jnp.float32),
    ],
)

_MAIN_CALL = pl.pallas_call(
    _nca_body,
    in_specs=[
        pl.BlockSpec(memory_space=pltpu.VMEM),
        pl.BlockSpec(memory_space=pltpu.SMEM),
        pl.BlockSpec(memory_space=pltpu.SMEM),
        pl.BlockSpec(memory_space=pltpu.SMEM),
        pl.BlockSpec(memory_space=pltpu.SMEM),
    ],
    out_shape=[
        jax.ShapeDtypeStruct((_NUM_CLASSES, _NN, _MM), jnp.float32),
        jax.ShapeDtypeStruct((_OUT_DIM, _NN, _MM), jnp.float32),
    ],
    scratch_shapes=[
        pltpu.VMEM((_ST, _N, _M), jnp.float32),
        pltpu.VMEM((3, _ST, _N, _M), jnp.float32),
        pltpu.VMEM((9, _N, _M), jnp.float32),
        pltpu.VMEM((_D_IN, _N, _M), jnp.bfloat16),
    ],
)


@jax.jit
def kernel(img, W1, b1, W2, b2):
    img_pad = jnp.pad(img[:, :, 0], 2)
    w1q, w2q = _PREP_CALL(W1, W2)
    cls_pl, g_pl = _MAIN_CALL(
        img_pad, w1q, w2q, b1.reshape(1, _D_IN), b2.reshape(1, _OUT_DIM)
    )
    cls_state = jnp.transpose(cls_pl, (1, 2, 0))
    guesses = jnp.transpose(g_pl, (1, 2, 0)).reshape(_NN * _MM, _OUT_DIM)
    return cls_state, guesses


# speedup vs baseline: 44.6358x; 7.3155x over previous
import jax
import jax.numpy as jnp
from jax.experimental import pallas as pl
from jax.experimental.pallas import tpu as pltpu

_NUM_CLASSES = 10
_ITERS = 3
_N = 256
_M = 256
_NN = 254
_MM = 254
_IN_DIM = 15
_OUT_DIM = 16
_D_IN = 137
_ST = 14


def _q(x):
    return x.astype(jnp.bfloat16).astype(jnp.float32)


def _prep_body(W1t_ref, W2t_ref, w1b_ref, w2b_ref):
    w1b_ref[:] = W1t_ref[:].astype(jnp.bfloat16)
    w2b_ref[:] = W2t_ref[:].astype(jnp.bfloat16)


def _nca_body(img_ref, w1b_ref, w2b_ref, b1_ref, b2_ref,
              cls_ref, g_ref, st_ref, stq_ref, feat_ref, b_ref):
    f32 = jnp.float32
    st_ref[...] = jnp.zeros(st_ref.shape, f32)
    stq_ref[...] = jnp.zeros(stq_ref.shape, f32)

    w1b = w1b_ref[:]
    w2b = w2b_ref[:]
    b1v = b1_ref[:]
    b2v = b2_ref[:]

    gxi = jax.lax.broadcasted_iota(jnp.int32, (_NN, _MM), 0)
    gyi = jax.lax.broadcasted_iota(jnp.int32, (_NN, _MM), 1)
    dxp = jnp.zeros((_NN, _MM), jnp.int32)
    dyp = jnp.zeros((_NN, _MM), jnp.int32)

    for t in range(_ITERS):
        if t == 0:
            for dx in range(3):
                for dy in range(3):
                    feat_ref[dx * 3 + dy, 0:_NN, 0:_MM] = _q(
                        img_ref[2 + dx:2 + dx + _NN, 2 + dy:2 + dy + _MM]
                    )
        else:
            offs = list(range(-t, t + 1))
            masks = []
            for ox in offs:
                mrow = dxp == ox
                for oy in offs:
                    masks.append(((mrow & (dyp == oy)).astype(f32), ox, oy))
            for dx in range(3):
                for dy in range(3):
                    acc = None
                    for m, ox, oy in masks:
                        term = m * img_ref[
                            2 + ox + dx:2 + ox + dx + _NN,
                            2 + oy + dy:2 + oy + dy + _MM,
                        ]
                        acc = term if acc is None else acc + term
                    feat_ref[dx * 3 + dy, 0:_NN, 0:_MM] = _q(acc)

        feat_ref[9, 0:_NN, 0:_MM] = _q(
            (gxi + dxp - _N // 2).astype(f32) * (1.0 / (_N // 2))
        )
        feat_ref[10, 0:_NN, 0:_MM] = _q(
            (gyi + dyp - _M // 2).astype(f32) * (1.0 / (_M // 2))
        )

        def row(i, _):
            for dx in range(3):
                for dy in range(3):
                    base = (dx * 3 + dy) * _IN_DIM
                    b_ref[base:base + 1, :] = feat_ref[dx * 3 + dy, pl.ds(i, 1), 0:_MM]
                    for c in range(_ST):
                        b_ref[base + 1 + c:base + 2 + c, :] = stq_ref[
                            c, pl.ds(i + dx, 1), dy:dy + _MM
                        ]
            b_ref[_D_IN - 2:_D_IN - 1, :] = feat_ref[9, pl.ds(i, 1), 0:_MM]
            b_ref[_D_IN - 1:_D_IN, :] = feat_ref[10, pl.ds(i, 1), 0:_MM]

            Bb = b_ref[:].astype(jnp.bfloat16)
            h = jnp.dot(w1b, Bb, preferred_element_type=f32) + b1v
            hq = h.astype(jnp.bfloat16)
            g_row = jnp.dot(w2b, hq, preferred_element_type=f32) + b2v
            for o in range(_OUT_DIM):
                g_ref[o, pl.ds(i, 1), :] = g_row[o:o + 1, :]
            return 0

        jax.lax.fori_loop(0, _NN, row, 0)

        if t < _ITERS - 1:
            thr = 0.0007
            a14 = g_ref[_OUT_DIM - 2]
            a15 = g_ref[_OUT_DIM - 1]
            ax = jnp.where(a14 < -thr, -1, jnp.where(a14 > thr, 1, 0))
            ay = jnp.where(a15 < -thr, -1, jnp.where(a15 > thr, 1, 0))
            dxp = jnp.clip(gxi + dxp + ax.astype(jnp.int32), 0, _N - 3) - gxi
            dyp = jnp.clip(gyi + dyp + ay.astype(jnp.int32), 0, _M - 3) - gyi

        if t < _ITERS - 1:

            def supd(c, _):
                st_ref[c, 1:1 + _NN, 1:1 + _MM] = (
                    st_ref[c, 1:1 + _NN, 1:1 + _MM] + g_ref[c]
                )
                stq_ref[c] = _q(st_ref[c])
                return 0

            jax.lax.fori_loop(0, _ST, supd, 0)
        else:

            def supd_last(c, _):
                st_ref[c, 1:1 + _NN, 1:1 + _MM] = (
                    st_ref[c, 1:1 + _NN, 1:1 + _MM] + g_ref[c]
                )
                return 0

            jax.lax.fori_loop(0, _ST, supd_last, 0)

    def cwrite(k, _):
        cls_ref[k] = st_ref[_ST - _NUM_CLASSES + k, 1:1 + _NN, 1:1 + _MM]
        return 0

    jax.lax.fori_loop(0, _NUM_CLASSES, cwrite, 0)


_PREP_CALL = pl.pallas_call(
    _prep_body,
    out_shape=[
        jax.ShapeDtypeStruct((_D_IN, _D_IN), jnp.bfloat16),
        jax.ShapeDtypeStruct((_OUT_DIM, _D_IN), jnp.bfloat16),
    ],
)

_MAIN_CALL = pl.pallas_call(
    _nca_body,
    out_shape=[
        jax.ShapeDtypeStruct((_NUM_CLASSES, _NN, _MM), jnp.float32),
        jax.ShapeDtypeStruct((_OUT_DIM, _NN, _MM), jnp.float32),
    ],
    scratch_shapes=[
        pltpu.VMEM((_ST, _N, _M), jnp.float32),
        pltpu.VMEM((_ST, _N, _M), jnp.float32),
        pltpu.VMEM((11, _N, _M), jnp.float32),
        pltpu.VMEM((_D_IN, _MM), jnp.float32),
    ],
)


@jax.jit
def kernel(img, W1, b1, W2, b2):
    img_pad = jnp.pad(img[:, :, 0], 2)
    w1b, w2b = _PREP_CALL(W1.T, W2.T)
    cls_pl, g_pl = _MAIN_CALL(
        img_pad, w1b, w2b, b1.reshape(_D_IN, 1), b2.reshape(_OUT_DIM, 1)
    )
    cls_state = jnp.transpose(cls_pl, (1, 2, 0))
    guesses = jnp.transpose(g_pl, (1, 2, 0)).reshape(_NN * _MM, _OUT_DIM)
    return cls_state, guesses


# speedup vs baseline: 48.9341x; 1.0963x over previous
import jax
import jax.numpy as jnp
from jax.experimental import pallas as pl
from jax.experimental.pallas import tpu as pltpu

_NUM_CLASSES = 10
_ITERS = 3
_N = 256
_M = 256
_NN = 254
_MM = 254
_IN_DIM = 15
_OUT_DIM = 16
_D_IN = 137
_ST = 14


def _q(x):
    return x.astype(jnp.bfloat16).astype(jnp.float32)


def _prep_body(W1t_ref, W2t_ref, w1b_ref, w2b_ref):
    w1b_ref[:] = W1t_ref[:].astype(jnp.bfloat16)
    w2b_ref[:] = W2t_ref[:].astype(jnp.bfloat16)


def _nca_body(img_ref, w1b_ref, w2b_ref, b1_ref, b2_ref,
              cls_ref, g_ref, st_ref, stq_ref, feat_ref, b_ref):
    f32 = jnp.float32
    st_ref[...] = jnp.zeros(st_ref.shape, f32)
    stq_ref[...] = jnp.zeros(stq_ref.shape, f32)
    b_ref[...] = jnp.zeros(b_ref.shape, f32)

    w1b = w1b_ref[:]
    w2b = w2b_ref[:]
    b1v = b1_ref[:]
    b2v = b2_ref[:]

    gxi = jax.lax.broadcasted_iota(jnp.int32, (_NN, _MM), 0)
    gyi = jax.lax.broadcasted_iota(jnp.int32, (_NN, _MM), 1)
    dxp = jnp.zeros((_NN, _MM), jnp.int32)
    dyp = jnp.zeros((_NN, _MM), jnp.int32)

    for t in range(_ITERS):
        if t == 0:
            for dx in range(3):
                for dy in range(3):
                    feat_ref[dx * 3 + dy, 0:_NN, 0:_MM] = _q(
                        img_ref[2 + dx:2 + dx + _NN, 2 + dy:2 + dy + _MM]
                    )
        else:
            offs = list(range(-t, t + 1))
            masks = []
            for ox in offs:
                mrow = dxp == ox
                for oy in offs:
                    masks.append(((mrow & (dyp == oy)).astype(f32), ox, oy))
            for dx in range(3):
                for dy in range(3):
                    acc = None
                    for m, ox, oy in masks:
                        term = m * img_ref[
                            2 + ox + dx:2 + ox + dx + _NN,
                            2 + oy + dy:2 + oy + dy + _MM,
                        ]
                        acc = term if acc is None else acc + term
                    feat_ref[dx * 3 + dy, 0:_NN, 0:_MM] = _q(acc)

        feat_ref[9, 0:_NN, 0:_MM] = _q(
            (gxi + dxp - _N // 2).astype(f32) * (1.0 / (_N // 2))
        )
        feat_ref[10, 0:_NN, 0:_MM] = _q(
            (gyi + dyp - _M // 2).astype(f32) * (1.0 / (_M // 2))
        )

        def row(i, _, t=t):
            p = jax.lax.rem(i, 2)
            for dx in range(3):
                for dy in range(3):
                    base = (dx * 3 + dy) * _IN_DIM
                    b_ref[p, base:base + 1, :] = feat_ref[
                        dx * 3 + dy, pl.ds(i, 1), 0:_MM
                    ]
                    if t > 0:
                        for c in range(_ST):
                            b_ref[p, base + 1 + c:base + 2 + c, :] = stq_ref[
                                c, pl.ds(i + dx, 1), dy:dy + _MM
                            ]
            b_ref[p, _D_IN - 2:_D_IN - 1, :] = feat_ref[9, pl.ds(i, 1), 0:_MM]
            b_ref[p, _D_IN - 1:_D_IN, :] = feat_ref[10, pl.ds(i, 1), 0:_MM]

            Bb = b_ref[p].astype(jnp.bfloat16)
            h = jnp.dot(w1b, Bb, preferred_element_type=f32) + b1v
            hq = h.astype(jnp.bfloat16)
            g_row = jnp.dot(w2b, hq, preferred_element_type=f32) + b2v
            for o in range(_OUT_DIM):
                g_ref[o, pl.ds(i, 1), :] = g_row[o:o + 1, :]
            return 0

        jax.lax.fori_loop(0, _NN, row, 0)

        if t < _ITERS - 1:
            thr = 0.0007
            a14 = g_ref[_OUT_DIM - 2]
            a15 = g_ref[_OUT_DIM - 1]
            ax = jnp.where(a14 < -thr, -1, jnp.where(a14 > thr, 1, 0))
            ay = jnp.where(a15 < -thr, -1, jnp.where(a15 > thr, 1, 0))
            dxp = jnp.clip(gxi + dxp + ax.astype(jnp.int32), 0, _N - 3) - gxi
            dyp = jnp.clip(gyi + dyp + ay.astype(jnp.int32), 0, _M - 3) - gyi

        if t < _ITERS - 1:

            def supd(c, _):
                st_ref[c, 1:1 + _NN, 1:1 + _MM] = (
                    st_ref[c, 1:1 + _NN, 1:1 + _MM] + g_ref[c]
                )
                stq_ref[c] = _q(st_ref[c])
                return 0

            jax.lax.fori_loop(0, _ST, supd, 0)
        else:

            def supd_last(c, _):
                st_ref[c, 1:1 + _NN, 1:1 + _MM] = (
                    st_ref[c, 1:1 + _NN, 1:1 + _MM] + g_ref[c]
                )
                return 0

            jax.lax.fori_loop(0, _ST, supd_last, 0)

    def cwrite(k, _):
        cls_ref[k] = st_ref[_ST - _NUM_CLASSES + k, 1:1 + _NN, 1:1 + _MM]
        return 0

    jax.lax.fori_loop(0, _NUM_CLASSES, cwrite, 0)


_PREP_CALL = pl.pallas_call(
    _prep_body,
    out_shape=[
        jax.ShapeDtypeStruct((_D_IN, _D_IN), jnp.bfloat16),
        jax.ShapeDtypeStruct((_OUT_DIM, _D_IN), jnp.bfloat16),
    ],
)

_MAIN_CALL = pl.pallas_call(
    _nca_body,
    out_shape=[
        jax.ShapeDtypeStruct((_NUM_CLASSES, _NN, _MM), jnp.float32),
        jax.ShapeDtypeStruct((_OUT_DIM, _NN, _MM), jnp.float32),
    ],
    scratch_shapes=[
        pltpu.VMEM((_ST, _N, _M), jnp.float32),
        pltpu.VMEM((_ST, _N, _M), jnp.float32),
        pltpu.VMEM((11, _N, _M), jnp.float32),
        pltpu.VMEM((2, _D_IN, _MM), jnp.float32),
    ],
)


@jax.jit
def kernel(img, W1, b1, W2, b2):
    img_pad = jnp.pad(img[:, :, 0], 2)
    w1b, w2b = _PREP_CALL(W1.T, W2.T)
    cls_pl, g_pl = _MAIN_CALL(
        img_pad, w1b, w2b, b1.reshape(_D_IN, 1), b2.reshape(_OUT_DIM, 1)
    )
    cls_state = jnp.transpose(cls_pl, (1, 2, 0))
    guesses = jnp.transpose(g_pl, (1, 2, 0)).reshape(_NN * _MM, _OUT_DIM)
    return cls_state, guesses
